# Initial kernel scaffold; baseline (speedup 1.0000x reference)
import jax
import jax.numpy as jnp
from jax.experimental import pallas as pl


def kernel(x, Wqkv, Wo, Wgate, Weg, Weu, Wed, ln1, ln2):
    raise NotImplementedError("write your pallas kernel here")


# speedup vs baseline: 2.7195x; 2.7195x over previous
import functools

import jax
import jax.numpy as jnp
from jax import lax
from jax.experimental import pallas as pl
from jax.experimental.pallas import tpu as pltpu
from jax.experimental.pallas import tpu_sc as plsc

EMB = 1024
HEADS = 16
HEAD_DIM = EMB // HEADS
FFN = EMB * 3
NUM_EXPERTS = 8
TOP_K = 2
EPS = 1e-6
SEQ = 2048
NSLOTS = SEQ * TOP_K
ROW_TILE = 512
N_TILES = NSLOTS // ROW_TILE
MAX_ITEMS = N_TILES + NUM_EXPERTS - 1
FFN_CHUNK = 1024
N_FCHUNK = FFN // FFN_CHUNK


def _qkv_body(x_ref, w_ref, ln1_ref, out_ref):
    xb = x_ref[...]
    ms = jnp.mean(xb * xb, axis=1, keepdims=True)
    h = xb * lax.rsqrt(ms + EPS) * ln1_ref[...]
    out_ref[...] = jnp.dot(h, w_ref[...], preferred_element_type=jnp.float32)


def _attn_body(q_ref, k_ref, v_ref, cq_ref, sq_ref, ck_ref, sk_ref, r_ref,
               out_ref):
    i = pl.program_id(1)
    qh = q_ref[0]
    kh = k_ref[0]
    vh = v_ref[0]
    rot = r_ref[...]
    qr = qh * cq_ref[...] + jnp.dot(qh, rot, preferred_element_type=jnp.float32) * sq_ref[...]
    kr = kh * ck_ref[...] + jnp.dot(kh, rot, preferred_element_type=jnp.float32) * sk_ref[...]
    s = lax.dot_general(qr, kr, (((1,), (1,)), ((), ())),
                        preferred_element_type=jnp.float32)
    s = s * (HEAD_DIM ** -0.5)
    rows = i * 512 + lax.broadcasted_iota(jnp.int32, s.shape, 0)
    cols = lax.broadcasted_iota(jnp.int32, s.shape, 1)
    s = jnp.where(cols > rows, -jnp.inf, s)
    mx = jnp.max(s, axis=1, keepdims=True)
    e = jnp.exp(s - mx)
    p = e / jnp.sum(e, axis=1, keepdims=True)
    out_ref[0] = jnp.dot(p, vh, preferred_element_type=jnp.float32)


def _proj_body(x_ref, ao_ref, wo_ref, ln2_ref, h1_ref, m_ref):
    h1 = x_ref[...] + jnp.dot(ao_ref[...], wo_ref[...],
                              preferred_element_type=jnp.float32)
    h1_ref[...] = h1
    ms = jnp.mean(h1 * h1, axis=1, keepdims=True)
    m_ref[...] = h1 * lax.rsqrt(ms + EPS) * ln2_ref[...]


def _router_body(m_ref, wg_ref, tri_ref, d0_ref, d1_ref, w0_ref,
                 w1_ref, off_ref, aux_ref):
    m = m_ref[...]
    logits = jnp.dot(m, wg_ref[...], preferred_element_type=jnp.float32)
    mx = jnp.max(logits, axis=1, keepdims=True)
    eg = jnp.exp(logits - mx)
    gate = eg / jnp.sum(eg, axis=1, keepdims=True)

    lanes = lax.broadcasted_iota(jnp.int32, gate.shape, 1)
    i1 = jnp.argmax(gate, axis=1).astype(jnp.int32)[:, None]
    w1v = jnp.max(gate, axis=1, keepdims=True)
    gate2 = jnp.where(lanes == i1, -jnp.inf, gate)
    i2 = jnp.argmax(gate2, axis=1).astype(jnp.int32)[:, None]
    w2v = jnp.max(gate2, axis=1, keepdims=True)
    t = jnp.exp(w2v - w1v)
    p1 = 1.0 / (1.0 + t)
    p2 = t / (1.0 + t)

    o1 = (lanes == i1).astype(jnp.float32)
    o2 = (lanes == i2).astype(jnp.float32)
    tri = tri_ref[...]
    c1 = jnp.dot(tri, o1, preferred_element_type=jnp.float32)
    c2 = jnp.dot(tri, o2, preferred_element_type=jnp.float32)
    rank1 = jnp.sum(o1 * c1, axis=1, keepdims=True) - 1.0
    rank2 = jnp.sum(o2 * c2, axis=1, keepdims=True) - 1.0
    cnt1 = c1[SEQ - 1:SEQ, :]
    cnt2 = c2[SEQ - 1:SEQ, :]
    counts = cnt1 + cnt2
    ue = lax.broadcasted_iota(jnp.int32, (NUM_EXPERTS, NUM_EXPERTS), 0)
    uc = lax.broadcasted_iota(jnp.int32, (NUM_EXPERTS, NUM_EXPERTS), 1)
    strict_upper = (ue < uc).astype(jnp.float32)
    cq = jnp.floor(counts * 0.0625)
    cl = counts - 16.0 * cq
    off = (16.0 * jnp.dot(cq, strict_upper, preferred_element_type=jnp.float32)
           + jnp.dot(cl, strict_upper, preferred_element_type=jnp.float32))
    dest1 = jnp.sum(o1 * off, axis=1, keepdims=True) + rank1
    dest2 = jnp.sum(o2 * (off + cnt1), axis=1, keepdims=True) + rank2

    zcol = jnp.zeros((1, 1), jnp.float32)

    def to_row(dcol):
        q = jnp.floor(dcol * 0.0625)
        lo = dcol - 16.0 * q
        rq = lax.dot_general(q, tri, (((0,), (0,)), ((), ())),
                             preferred_element_type=jnp.float32)
        rl = lax.dot_general(lo, tri, (((0,), (0,)), ((), ())),
                             preferred_element_type=jnp.float32)
        r = 16.0 * rq + rl
        sh = jnp.concatenate([r[:, 1:], zcol], axis=1)
        return r - sh

    d0_ref[...] = jnp.broadcast_to(to_row(dest1).astype(jnp.int32), (8, SEQ))
    d1_ref[...] = jnp.broadcast_to(to_row(dest2).astype(jnp.int32), (8, SEQ))
    w0_ref[...] = jnp.broadcast_to(p1, (SEQ, 128))
    w1_ref[...] = jnp.broadcast_to(p2, (SEQ, 128))

    eye8 = (ue == uc).astype(jnp.float32)
    oq = jnp.floor(off * 0.0625)
    ol = off - 16.0 * oq
    offcol = (16.0 * lax.dot_general(eye8, oq, (((1,), (1,)), ((), ())),
                                     preferred_element_type=jnp.float32)
              + lax.dot_general(eye8, ol, (((1,), (1,)), ((), ())),
                                preferred_element_type=jnp.float32))
    off_ref[...] = jnp.broadcast_to(offcol.astype(jnp.int32),
                                    (NUM_EXPERTS, 128))

    importance = jnp.mean(gate, axis=0, keepdims=True)
    load = counts / jnp.float32(SEQ)
    aux = jnp.float32(NUM_EXPERTS) * jnp.sum(importance * load)
    aux_ref[...] = jnp.broadcast_to(aux[None, None], (8, 128))


def _ffn_body(tid_ref, eid_ref, rs_ref, re_ref, ff_ref,
              xs_ref, weg_ref, weu_ref, wed_ref, out_ref):
    s = pl.program_id(0)
    f = pl.program_id(1)
    xb = xs_ref[...]
    g = jnp.dot(xb, weg_ref[0], preferred_element_type=jnp.float32)
    u = jnp.dot(xb, weu_ref[0], preferred_element_type=jnp.float32)
    h = (g * (1.0 / (1.0 + jnp.exp(-g)))) * u
    grow = tid_ref[s] * ROW_TILE + lax.broadcasted_iota(
        jnp.int32, (ROW_TILE, 1), 0)
    maskv = (grow >= rs_ref[s]) & (grow < re_ref[s])
    h = jnp.where(maskv, h, 0.0)
    contrib = jnp.dot(h, wed_ref[0], preferred_element_type=jnp.float32)
    is_first = (ff_ref[s] == 1) & (f == 0)

    @pl.when(is_first)
    def _():
        out_ref[...] = contrib

    @pl.when(jnp.logical_not(is_first))
    def _():
        out_ref[...] += contrib


def _combine_body(h1_ref, ga_ref, gb_ref, w0_ref, w1_ref, out_ref):
    w0 = w0_ref[...][:, 0:1]
    w1 = w1_ref[...][:, 0:1]
    out_ref[...] = h1_ref[...] + w0 * ga_ref[...] + w1 * gb_ref[...]


def _qkv_call(x2, Wqkv, ln1row):
    return pl.pallas_call(
        _qkv_body,
        grid=(SEQ // 512,),
        in_specs=[
            pl.BlockSpec((512, EMB), lambda i: (i, 0)),
            pl.BlockSpec((EMB, 3 * EMB), lambda i: (0, 0)),
            pl.BlockSpec((1, EMB), lambda i: (0, 0)),
        ],
        out_specs=pl.BlockSpec((512, 3 * EMB), lambda i: (i, 0)),
        out_shape=jax.ShapeDtypeStruct((SEQ, 3 * EMB), jnp.float32),
    )(x2, Wqkv, ln1row)


def _attn_call(q, k, v, cos, sin, rot):
    return pl.pallas_call(
        _attn_body,
        grid=(HEADS, SEQ // 512),
        in_specs=[
            pl.BlockSpec((1, 512, HEAD_DIM), lambda h, i: (h, i, 0)),
            pl.BlockSpec((1, SEQ, HEAD_DIM), lambda h, i: (h, 0, 0)),
            pl.BlockSpec((1, SEQ, HEAD_DIM), lambda h, i: (h, 0, 0)),
            pl.BlockSpec((512, HEAD_DIM), lambda h, i: (i, 0)),
            pl.BlockSpec((512, HEAD_DIM), lambda h, i: (i, 0)),
            pl.BlockSpec((SEQ, HEAD_DIM), lambda h, i: (0, 0)),
            pl.BlockSpec((SEQ, HEAD_DIM), lambda h, i: (0, 0)),
            pl.BlockSpec((HEAD_DIM, HEAD_DIM), lambda h, i: (0, 0)),
        ],
        out_specs=pl.BlockSpec((1, 512, HEAD_DIM), lambda h, i: (h, i, 0)),
        out_shape=jax.ShapeDtypeStruct((HEADS, SEQ, HEAD_DIM), jnp.float32),
    )(q, k, v, cos, sin, cos, sin, rot)


def _proj_call(x2, ao, Wo, ln2row):
    return pl.pallas_call(
        _proj_body,
        grid=(SEQ // 512,),
        in_specs=[
            pl.BlockSpec((512, EMB), lambda i: (i, 0)),
            pl.BlockSpec((512, EMB), lambda i: (i, 0)),
            pl.BlockSpec((EMB, EMB), lambda i: (0, 0)),
            pl.BlockSpec((1, EMB), lambda i: (0, 0)),
        ],
        out_specs=[
            pl.BlockSpec((512, EMB), lambda i: (i, 0)),
            pl.BlockSpec((512, EMB), lambda i: (i, 0)),
        ],
        out_shape=[
            jax.ShapeDtypeStruct((SEQ, EMB), jnp.float32),
            jax.ShapeDtypeStruct((SEQ, EMB), jnp.float32),
        ],
    )(x2, ao, Wo, ln2row)


def _router_call(m, Wgate, tri):
    return pl.pallas_call(
        _router_body,
        grid=(1,),
        in_specs=[
            pl.BlockSpec((SEQ, EMB), lambda i: (0, 0)),
            pl.BlockSpec((EMB, NUM_EXPERTS), lambda i: (0, 0)),
            pl.BlockSpec((SEQ, SEQ), lambda i: (0, 0)),
        ],
        out_specs=[
            pl.BlockSpec((8, SEQ), lambda i: (0, 0)),
            pl.BlockSpec((8, SEQ), lambda i: (0, 0)),
            pl.BlockSpec((SEQ, 128), lambda i: (0, 0)),
            pl.BlockSpec((SEQ, 128), lambda i: (0, 0)),
            pl.BlockSpec((NUM_EXPERTS, 128), lambda i: (0, 0)),
            pl.BlockSpec((8, 128), lambda i: (0, 0)),
        ],
        out_shape=[
            jax.ShapeDtypeStruct((8, SEQ), jnp.int32),
            jax.ShapeDtypeStruct((8, SEQ), jnp.int32),
            jax.ShapeDtypeStruct((SEQ, 128), jnp.float32),
            jax.ShapeDtypeStruct((SEQ, 128), jnp.float32),
            jax.ShapeDtypeStruct((NUM_EXPERTS, 128), jnp.int32),
            jax.ShapeDtypeStruct((8, 128), jnp.float32),
        ],
    )(m, Wgate, tri)


def _ffn_call(tile_id, expert_id, row_start, row_end, first_flag,
              xs, Weg, Weu, Wed):
    grid_spec = pltpu.PrefetchScalarGridSpec(
        num_scalar_prefetch=5,
        grid=(MAX_ITEMS, N_FCHUNK),
        in_specs=[
            pl.BlockSpec((ROW_TILE, EMB),
                         lambda s, f, tid, eid, rs, re, ff: (tid[s], 0)),
            pl.BlockSpec((1, EMB, FFN_CHUNK),
                         lambda s, f, tid, eid, rs, re, ff: (eid[s], 0, f)),
            pl.BlockSpec((1, EMB, FFN_CHUNK),
                         lambda s, f, tid, eid, rs, re, ff: (eid[s], 0, f)),
            pl.BlockSpec((1, FFN_CHUNK, EMB),
                         lambda s, f, tid, eid, rs, re, ff: (eid[s], f, 0)),
        ],
        out_specs=pl.BlockSpec((ROW_TILE, EMB),
                               lambda s, f, tid, eid, rs, re, ff: (tid[s], 0)),
    )
    return pl.pallas_call(
        _ffn_body,
        grid_spec=grid_spec,
        out_shape=jax.ShapeDtypeStruct((NSLOTS, EMB), jnp.float32),
    )(tile_id, expert_id, row_start, row_end, first_flag, xs, Weg, Weu, Wed)


def _combine_call(h1, ga, gb, w0b, w1b):
    return pl.pallas_call(
        _combine_body,
        grid=(SEQ // 512,),
        in_specs=[
            pl.BlockSpec((512, EMB), lambda i: (i, 0)),
            pl.BlockSpec((512, EMB), lambda i: (i, 0)),
            pl.BlockSpec((512, EMB), lambda i: (i, 0)),
            pl.BlockSpec((512, 128), lambda i: (i, 0)),
            pl.BlockSpec((512, 128), lambda i: (i, 0)),
        ],
        out_specs=pl.BlockSpec((512, EMB), lambda i: (i, 0)),
        out_shape=jax.ShapeDtypeStruct((SEQ, EMB), jnp.float32),
    )(h1, ga, gb, w0b, w1b)


_NW = 32
_CHUNK = SEQ // _NW


@functools.lru_cache(maxsize=None)
def _sc_kernels():
    mesh = plsc.VectorSubcoreMesh(core_axis_name="c", subcore_axis_name="s")
    scratch = [
        pltpu.VMEM((_CHUNK, EMB), jnp.float32),
        pltpu.VMEM((_CHUNK,), jnp.int32),
        pltpu.SemaphoreType.DMA,
    ]

    @functools.partial(
        pl.kernel,
        out_type=jax.ShapeDtypeStruct((NSLOTS, EMB), jnp.float32),
        mesh=mesh,
        scratch_types=list(scratch),
    )
    def dispatch(m_hbm, d0_hbm, d1_hbm, xs_hbm, rows_v, idx_v, sem):
        wid = lax.axis_index("s") * 2 + lax.axis_index("c")
        base = wid * _CHUNK
        pltpu.sync_copy(m_hbm.at[pl.ds(base, _CHUNK)], rows_v)
        pltpu.sync_copy(d0_hbm.at[0, pl.ds(base, _CHUNK)], idx_v)
        pltpu.async_copy(rows_v, xs_hbm.at[idx_v], sem).wait()
        pltpu.sync_copy(d1_hbm.at[0, pl.ds(base, _CHUNK)], idx_v)
        pltpu.async_copy(rows_v, xs_hbm.at[idx_v], sem).wait()

    @functools.partial(
        pl.kernel,
        out_type=(jax.ShapeDtypeStruct((SEQ, EMB), jnp.float32),
                  jax.ShapeDtypeStruct((SEQ, EMB), jnp.float32)),
        mesh=mesh,
        scratch_types=list(scratch),
    )
    def gather(ffn_hbm, d0_hbm, d1_hbm, ga_hbm, gb_hbm, rows_v, idx_v, sem):
        wid = lax.axis_index("s") * 2 + lax.axis_index("c")
        base = wid * _CHUNK
        pltpu.sync_copy(d0_hbm.at[0, pl.ds(base, _CHUNK)], idx_v)
        pltpu.async_copy(ffn_hbm.at[idx_v], rows_v, sem).wait()
        pltpu.sync_copy(rows_v, ga_hbm.at[pl.ds(base, _CHUNK)])
        pltpu.sync_copy(d1_hbm.at[0, pl.ds(base, _CHUNK)], idx_v)
        pltpu.async_copy(ffn_hbm.at[idx_v], rows_v, sem).wait()
        pltpu.sync_copy(rows_v, gb_hbm.at[pl.ds(base, _CHUNK)])

    return dispatch, gather


def _dispatch_sc(m, d0b, d1b):
    return _sc_kernels()[0](m, d0b, d1b)


def _gather_sc(ffn, d0b, d1b):
    return _sc_kernels()[1](ffn, d0b, d1b)


def _rope_tables():
    inv = 1.0 / (10000.0 ** (jnp.arange(0, HEAD_DIM, 2, dtype=jnp.float32)
                             / HEAD_DIM))
    pos = jnp.arange(SEQ, dtype=jnp.float32)
    freqs = jnp.outer(pos, inv)
    emb = jnp.concatenate([freqs, freqs], axis=-1)
    return jnp.cos(emb), jnp.sin(emb)


def _rot_matrix():
    i = jnp.arange(HEAD_DIM)
    plus = (i[:, None] + 1 == i[None, :]) & (i[None, :] % 2 == 1)
    minus = (i[:, None] - 1 == i[None, :]) & (i[None, :] % 2 == 0)
    return plus.astype(jnp.float32) - minus.astype(jnp.float32)


def _tri_matrix():
    r = lax.broadcasted_iota(jnp.int32, (SEQ, SEQ), 0)
    c = lax.broadcasted_iota(jnp.int32, (SEQ, SEQ), 1)
    return (c <= r).astype(jnp.float32)


def _work_items(starts):
    i32 = jnp.int32
    ends = jnp.concatenate([starts[1:], jnp.array([NSLOTS], i32)])
    nonempty = ends > starts
    first_t = starts // ROW_TILE
    last_t = jnp.where(nonempty, (ends - 1) // ROW_TILE, first_t)
    ntiles = jnp.where(nonempty, last_t - first_t + 1, 0)
    base = jnp.concatenate([jnp.zeros((1,), i32), jnp.cumsum(ntiles)])
    s = jnp.arange(MAX_ITEMS, dtype=i32)
    e = jnp.sum((s[:, None] >= base[None, 1:]).astype(i32), axis=1)
    e = jnp.clip(e, 0, NUM_EXPERTS - 1)
    valid = s < base[NUM_EXPERTS]
    tile = first_t[e] + (s - base[e])
    tile_id = jnp.where(valid, tile, N_TILES - 1).astype(i32)
    expert_id = jnp.where(valid, e, 0).astype(i32)
    row_start = jnp.where(valid, starts[e], 0).astype(i32)
    row_end = jnp.where(valid, ends[e], 0).astype(i32)
    first_flag = jnp.concatenate(
        [jnp.ones((1,), i32),
         (tile_id[1:] != tile_id[:-1]).astype(i32)])
    return tile_id, expert_id, row_start, row_end, first_flag


def kernel(x, Wqkv, Wo, Wgate, Weg, Weu, Wed, ln1, ln2):
    B, S, D = x.shape
    x2 = x.reshape(S, D)
    cos, sin = _rope_tables()
    rot = _rot_matrix()
    tri = _tri_matrix()

    qkv = _qkv_call(x2, Wqkv, ln1.reshape(1, EMB))
    q = qkv[:, :EMB].reshape(S, HEADS, HEAD_DIM).transpose(1, 0, 2)
    k = qkv[:, EMB:2 * EMB].reshape(S, HEADS, HEAD_DIM).transpose(1, 0, 2)
    v = qkv[:, 2 * EMB:].reshape(S, HEADS, HEAD_DIM).transpose(1, 0, 2)
    ao = _attn_call(q, k, v, cos, sin, rot)
    ao2 = ao.transpose(1, 0, 2).reshape(S, D)
    h1, m = _proj_call(x2, ao2, Wo, ln2.reshape(1, EMB))

    d0b, d1b, w0b, w1b, offb, auxb = _router_call(m, Wgate, tri)
    starts = offb[:, 0]
    tile_id, expert_id, row_start, row_end, first_flag = _work_items(starts)

    xs = _dispatch_sc(m, d0b, d1b)
    ffn = _ffn_call(tile_id, expert_id, row_start, row_end, first_flag,
                    xs, Weg, Weu, Wed)
    ga, gb = _gather_sc(ffn, d0b, d1b)
    out = _combine_call(h1, ga, gb, w0b, w1b)

    aux = auxb[0, 0]
    return out.reshape(B, S, D), aux


# speedup vs baseline: 2.7245x; 1.0018x over previous
import functools

import jax
import jax.numpy as jnp
from jax import lax
from jax.experimental import pallas as pl
from jax.experimental.pallas import tpu as pltpu
from jax.experimental.pallas import tpu_sc as plsc

EMB = 1024
HEADS = 16
HEAD_DIM = EMB // HEADS
FFN = EMB * 3
NUM_EXPERTS = 8
TOP_K = 2
EPS = 1e-6
SEQ = 2048
NSLOTS = SEQ * TOP_K
ROW_TILE = 512
N_TILES = NSLOTS // ROW_TILE
MAX_ITEMS = N_TILES + NUM_EXPERTS - 1
FFN_CHUNK = 1024
N_FCHUNK = FFN // FFN_CHUNK


def _qkv_body(x_ref, w_ref, ln1_ref, out_ref):
    xb = x_ref[...]
    ms = jnp.mean(xb * xb, axis=1, keepdims=True)
    h = xb * lax.rsqrt(ms + EPS) * ln1_ref[...]
    out_ref[...] = jnp.dot(h, w_ref[...], preferred_element_type=jnp.float32)


def _attn_body(q_ref, k_ref, v_ref, cq_ref, sq_ref, ck_ref, sk_ref, r_ref,
               out_ref):
    i = pl.program_id(1)
    qh = q_ref[0]
    kh = k_ref[0]
    vh = v_ref[0]
    rot = r_ref[...]
    qr = qh * cq_ref[...] + jnp.dot(qh, rot, preferred_element_type=jnp.float32) * sq_ref[...]
    kr = kh * ck_ref[...] + jnp.dot(kh, rot, preferred_element_type=jnp.float32) * sk_ref[...]
    s = lax.dot_general(qr, kr, (((1,), (1,)), ((), ())),
                        preferred_element_type=jnp.float32)
    s = s * (HEAD_DIM ** -0.5)
    rows = i * 512 + lax.broadcasted_iota(jnp.int32, s.shape, 0)
    cols = lax.broadcasted_iota(jnp.int32, s.shape, 1)
    s = jnp.where(cols > rows, -jnp.inf, s)
    mx = jnp.max(s, axis=1, keepdims=True)
    e = jnp.exp(s - mx)
    p = e / jnp.sum(e, axis=1, keepdims=True)
    out_ref[0] = jnp.dot(p, vh, preferred_element_type=jnp.float32)


def _proj_body(x_ref, ao_ref, wo_ref, ln2_ref, h1_ref, m_ref):
    h1 = x_ref[...] + jnp.dot(ao_ref[...], wo_ref[...],
                              preferred_element_type=jnp.float32)
    h1_ref[...] = h1
    ms = jnp.mean(h1 * h1, axis=1, keepdims=True)
    m_ref[...] = h1 * lax.rsqrt(ms + EPS) * ln2_ref[...]


def _router_body(m_ref, wg_ref, tri_ref, d0_ref, d1_ref, w0_ref,
                 w1_ref, off_ref, aux_ref):
    m = m_ref[...]
    logits = jnp.dot(m, wg_ref[...], preferred_element_type=jnp.float32)
    mx = jnp.max(logits, axis=1, keepdims=True)
    eg = jnp.exp(logits - mx)
    gate = eg / jnp.sum(eg, axis=1, keepdims=True)

    lanes = lax.broadcasted_iota(jnp.int32, gate.shape, 1)
    i1 = jnp.argmax(gate, axis=1).astype(jnp.int32)[:, None]
    w1v = jnp.max(gate, axis=1, keepdims=True)
    gate2 = jnp.where(lanes == i1, -jnp.inf, gate)
    i2 = jnp.argmax(gate2, axis=1).astype(jnp.int32)[:, None]
    w2v = jnp.max(gate2, axis=1, keepdims=True)
    t = jnp.exp(w2v - w1v)
    p1 = 1.0 / (1.0 + t)
    p2 = t / (1.0 + t)

    o1 = (lanes == i1).astype(jnp.float32)
    o2 = (lanes == i2).astype(jnp.float32)
    tri = tri_ref[...]
    c1 = jnp.dot(tri, o1, preferred_element_type=jnp.float32)
    c2 = jnp.dot(tri, o2, preferred_element_type=jnp.float32)
    rank1 = jnp.sum(o1 * c1, axis=1, keepdims=True) - 1.0
    rank2 = jnp.sum(o2 * c2, axis=1, keepdims=True) - 1.0
    cnt1 = c1[SEQ - 1:SEQ, :]
    cnt2 = c2[SEQ - 1:SEQ, :]
    counts = cnt1 + cnt2
    ue = lax.broadcasted_iota(jnp.int32, (NUM_EXPERTS, NUM_EXPERTS), 0)
    uc = lax.broadcasted_iota(jnp.int32, (NUM_EXPERTS, NUM_EXPERTS), 1)
    strict_upper = (ue < uc).astype(jnp.float32)
    cq = jnp.floor(counts * 0.0625)
    cl = counts - 16.0 * cq
    off = (16.0 * jnp.dot(cq, strict_upper, preferred_element_type=jnp.float32)
           + jnp.dot(cl, strict_upper, preferred_element_type=jnp.float32))
    dest1 = jnp.sum(o1 * off, axis=1, keepdims=True) + rank1
    dest2 = jnp.sum(o2 * (off + cnt1), axis=1, keepdims=True) + rank2

    zcol = jnp.zeros((1, 1), jnp.float32)

    def to_row(dcol):
        q = jnp.floor(dcol * 0.0625)
        lo = dcol - 16.0 * q
        rq = lax.dot_general(q, tri, (((0,), (0,)), ((), ())),
                             preferred_element_type=jnp.float32)
        rl = lax.dot_general(lo, tri, (((0,), (0,)), ((), ())),
                             preferred_element_type=jnp.float32)
        r = 16.0 * rq + rl
        sh = jnp.concatenate([r[:, 1:], zcol], axis=1)
        return r - sh

    d0_ref[...] = jnp.broadcast_to(to_row(dest1).astype(jnp.int32), (8, SEQ))
    d1_ref[...] = jnp.broadcast_to(to_row(dest2).astype(jnp.int32), (8, SEQ))
    w0_ref[...] = jnp.broadcast_to(p1, (SEQ, 128))
    w1_ref[...] = jnp.broadcast_to(p2, (SEQ, 128))

    eye8 = (ue == uc).astype(jnp.float32)
    oq = jnp.floor(off * 0.0625)
    ol = off - 16.0 * oq
    offcol = (16.0 * lax.dot_general(eye8, oq, (((1,), (1,)), ((), ())),
                                     preferred_element_type=jnp.float32)
              + lax.dot_general(eye8, ol, (((1,), (1,)), ((), ())),
                                preferred_element_type=jnp.float32))
    off_ref[...] = jnp.broadcast_to(offcol.astype(jnp.int32),
                                    (NUM_EXPERTS, 128))

    importance = jnp.mean(gate, axis=0, keepdims=True)
    load = counts / jnp.float32(SEQ)
    aux = jnp.float32(NUM_EXPERTS) * jnp.sum(importance * load)
    aux_ref[...] = jnp.broadcast_to(aux[None, None], (8, 128))


def _ffn_body(tid_ref, eid_ref, rs_ref, re_ref, ff_ref,
              xs_ref, weg_ref, weu_ref, wed_ref, out_ref):
    s = pl.program_id(0)
    f = pl.program_id(1)
    xb = xs_ref[...].astype(jnp.bfloat16)
    g = jnp.dot(xb, weg_ref[0].astype(jnp.bfloat16),
                preferred_element_type=jnp.float32)
    u = jnp.dot(xb, weu_ref[0].astype(jnp.bfloat16),
                preferred_element_type=jnp.float32)
    h = (g * (1.0 / (1.0 + jnp.exp(-g)))) * u
    grow = tid_ref[s] * ROW_TILE + lax.broadcasted_iota(
        jnp.int32, (ROW_TILE, 1), 0)
    maskv = (grow >= rs_ref[s]) & (grow < re_ref[s])
    h = jnp.where(maskv, h, 0.0).astype(jnp.bfloat16)
    contrib = jnp.dot(h, wed_ref[0].astype(jnp.bfloat16),
                      preferred_element_type=jnp.float32)
    is_first = (ff_ref[s] == 1) & (f == 0)

    @pl.when(is_first)
    def _():
        out_ref[...] = contrib

    @pl.when(jnp.logical_not(is_first))
    def _():
        out_ref[...] += contrib


def _combine_body(h1_ref, ga_ref, gb_ref, w0_ref, w1_ref, out_ref):
    w0 = w0_ref[...][:, 0:1]
    w1 = w1_ref[...][:, 0:1]
    out_ref[...] = h1_ref[...] + w0 * ga_ref[...] + w1 * gb_ref[...]


def _qkv_call(x2, Wqkv, ln1row):
    return pl.pallas_call(
        _qkv_body,
        grid=(SEQ // 512,),
        in_specs=[
            pl.BlockSpec((512, EMB), lambda i: (i, 0)),
            pl.BlockSpec((EMB, 3 * EMB), lambda i: (0, 0)),
            pl.BlockSpec((1, EMB), lambda i: (0, 0)),
        ],
        out_specs=pl.BlockSpec((512, 3 * EMB), lambda i: (i, 0)),
        out_shape=jax.ShapeDtypeStruct((SEQ, 3 * EMB), jnp.float32),
    )(x2, Wqkv, ln1row)


def _attn_call(q, k, v, cos, sin, rot):
    return pl.pallas_call(
        _attn_body,
        grid=(HEADS, SEQ // 512),
        in_specs=[
            pl.BlockSpec((1, 512, HEAD_DIM), lambda h, i: (h, i, 0)),
            pl.BlockSpec((1, SEQ, HEAD_DIM), lambda h, i: (h, 0, 0)),
            pl.BlockSpec((1, SEQ, HEAD_DIM), lambda h, i: (h, 0, 0)),
            pl.BlockSpec((512, HEAD_DIM), lambda h, i: (i, 0)),
            pl.BlockSpec((512, HEAD_DIM), lambda h, i: (i, 0)),
            pl.BlockSpec((SEQ, HEAD_DIM), lambda h, i: (0, 0)),
            pl.BlockSpec((SEQ, HEAD_DIM), lambda h, i: (0, 0)),
            pl.BlockSpec((HEAD_DIM, HEAD_DIM), lambda h, i: (0, 0)),
        ],
        out_specs=pl.BlockSpec((1, 512, HEAD_DIM), lambda h, i: (h, i, 0)),
        out_shape=jax.ShapeDtypeStruct((HEADS, SEQ, HEAD_DIM), jnp.float32),
    )(q, k, v, cos, sin, cos, sin, rot)


def _proj_call(x2, ao, Wo, ln2row):
    return pl.pallas_call(
        _proj_body,
        grid=(SEQ // 512,),
        in_specs=[
            pl.BlockSpec((512, EMB), lambda i: (i, 0)),
            pl.BlockSpec((512, EMB), lambda i: (i, 0)),
            pl.BlockSpec((EMB, EMB), lambda i: (0, 0)),
            pl.BlockSpec((1, EMB), lambda i: (0, 0)),
        ],
        out_specs=[
            pl.BlockSpec((512, EMB), lambda i: (i, 0)),
            pl.BlockSpec((512, EMB), lambda i: (i, 0)),
        ],
        out_shape=[
            jax.ShapeDtypeStruct((SEQ, EMB), jnp.float32),
            jax.ShapeDtypeStruct((SEQ, EMB), jnp.float32),
        ],
    )(x2, ao, Wo, ln2row)


def _router_call(m, Wgate, tri):
    return pl.pallas_call(
        _router_body,
        grid=(1,),
        in_specs=[
            pl.BlockSpec((SEQ, EMB), lambda i: (0, 0)),
            pl.BlockSpec((EMB, NUM_EXPERTS), lambda i: (0, 0)),
            pl.BlockSpec((SEQ, SEQ), lambda i: (0, 0)),
        ],
        out_specs=[
            pl.BlockSpec((8, SEQ), lambda i: (0, 0)),
            pl.BlockSpec((8, SEQ), lambda i: (0, 0)),
            pl.BlockSpec((SEQ, 128), lambda i: (0, 0)),
            pl.BlockSpec((SEQ, 128), lambda i: (0, 0)),
            pl.BlockSpec((NUM_EXPERTS, 128), lambda i: (0, 0)),
            pl.BlockSpec((8, 128), lambda i: (0, 0)),
        ],
        out_shape=[
            jax.ShapeDtypeStruct((8, SEQ), jnp.int32),
            jax.ShapeDtypeStruct((8, SEQ), jnp.int32),
            jax.ShapeDtypeStruct((SEQ, 128), jnp.float32),
            jax.ShapeDtypeStruct((SEQ, 128), jnp.float32),
            jax.ShapeDtypeStruct((NUM_EXPERTS, 128), jnp.int32),
            jax.ShapeDtypeStruct((8, 128), jnp.float32),
        ],
    )(m, Wgate, tri)


def _ffn_call(tile_id, expert_id, row_start, row_end, first_flag,
              xs, Weg, Weu, Wed):
    grid_spec = pltpu.PrefetchScalarGridSpec(
        num_scalar_prefetch=5,
        grid=(MAX_ITEMS, N_FCHUNK),
        in_specs=[
            pl.BlockSpec((ROW_TILE, EMB),
                         lambda s, f, tid, eid, rs, re, ff: (tid[s], 0)),
            pl.BlockSpec((1, EMB, FFN_CHUNK),
                         lambda s, f, tid, eid, rs, re, ff: (eid[s], 0, f)),
            pl.BlockSpec((1, EMB, FFN_CHUNK),
                         lambda s, f, tid, eid, rs, re, ff: (eid[s], 0, f)),
            pl.BlockSpec((1, FFN_CHUNK, EMB),
                         lambda s, f, tid, eid, rs, re, ff: (eid[s], f, 0)),
        ],
        out_specs=pl.BlockSpec((ROW_TILE, EMB),
                               lambda s, f, tid, eid, rs, re, ff: (tid[s], 0)),
    )
    return pl.pallas_call(
        _ffn_body,
        grid_spec=grid_spec,
        out_shape=jax.ShapeDtypeStruct((NSLOTS, EMB), jnp.float32),
    )(tile_id, expert_id, row_start, row_end, first_flag, xs, Weg, Weu, Wed)


def _combine_call(h1, ga, gb, w0b, w1b):
    return pl.pallas_call(
        _combine_body,
        grid=(SEQ // 512,),
        in_specs=[
            pl.BlockSpec((512, EMB), lambda i: (i, 0)),
            pl.BlockSpec((512, EMB), lambda i: (i, 0)),
            pl.BlockSpec((512, EMB), lambda i: (i, 0)),
            pl.BlockSpec((512, 128), lambda i: (i, 0)),
            pl.BlockSpec((512, 128), lambda i: (i, 0)),
        ],
        out_specs=pl.BlockSpec((512, EMB), lambda i: (i, 0)),
        out_shape=jax.ShapeDtypeStruct((SEQ, EMB), jnp.float32),
    )(h1, ga, gb, w0b, w1b)


_NW = 32
_CHUNK = SEQ // _NW


@functools.lru_cache(maxsize=None)
def _sc_kernels():
    mesh = plsc.VectorSubcoreMesh(core_axis_name="c", subcore_axis_name="s")
    scratch = [
        pltpu.VMEM((_CHUNK, EMB), jnp.float32),
        pltpu.VMEM((_CHUNK,), jnp.int32),
        pltpu.SemaphoreType.DMA,
    ]

    @functools.partial(
        pl.kernel,
        out_type=jax.ShapeDtypeStruct((NSLOTS, EMB), jnp.float32),
        mesh=mesh,
        scratch_types=list(scratch),
    )
    def dispatch(m_hbm, d0_hbm, d1_hbm, xs_hbm, rows_v, idx_v, sem):
        wid = lax.axis_index("s") * 2 + lax.axis_index("c")
        base = wid * _CHUNK
        pltpu.sync_copy(m_hbm.at[pl.ds(base, _CHUNK)], rows_v)
        pltpu.sync_copy(d0_hbm.at[0, pl.ds(base, _CHUNK)], idx_v)
        pltpu.async_copy(rows_v, xs_hbm.at[idx_v], sem).wait()
        pltpu.sync_copy(d1_hbm.at[0, pl.ds(base, _CHUNK)], idx_v)
        pltpu.async_copy(rows_v, xs_hbm.at[idx_v], sem).wait()

    @functools.partial(
        pl.kernel,
        out_type=(jax.ShapeDtypeStruct((SEQ, EMB), jnp.float32),
                  jax.ShapeDtypeStruct((SEQ, EMB), jnp.float32)),
        mesh=mesh,
        scratch_types=list(scratch),
    )
    def gather(ffn_hbm, d0_hbm, d1_hbm, ga_hbm, gb_hbm, rows_v, idx_v, sem):
        wid = lax.axis_index("s") * 2 + lax.axis_index("c")
        base = wid * _CHUNK
        pltpu.sync_copy(d0_hbm.at[0, pl.ds(base, _CHUNK)], idx_v)
        pltpu.async_copy(ffn_hbm.at[idx_v], rows_v, sem).wait()
        pltpu.sync_copy(rows_v, ga_hbm.at[pl.ds(base, _CHUNK)])
        pltpu.sync_copy(d1_hbm.at[0, pl.ds(base, _CHUNK)], idx_v)
        pltpu.async_copy(ffn_hbm.at[idx_v], rows_v, sem).wait()
        pltpu.sync_copy(rows_v, gb_hbm.at[pl.ds(base, _CHUNK)])

    return dispatch, gather


def _dispatch_sc(m, d0b, d1b):
    return _sc_kernels()[0](m, d0b, d1b)


def _gather_sc(ffn, d0b, d1b):
    return _sc_kernels()[1](ffn, d0b, d1b)


def _rope_tables():
    inv = 1.0 / (10000.0 ** (jnp.arange(0, HEAD_DIM, 2, dtype=jnp.float32)
                             / HEAD_DIM))
    pos = jnp.arange(SEQ, dtype=jnp.float32)
    freqs = jnp.outer(pos, inv)
    emb = jnp.concatenate([freqs, freqs], axis=-1)
    return jnp.cos(emb), jnp.sin(emb)


def _rot_matrix():
    i = jnp.arange(HEAD_DIM)
    plus = (i[:, None] + 1 == i[None, :]) & (i[None, :] % 2 == 1)
    minus = (i[:, None] - 1 == i[None, :]) & (i[None, :] % 2 == 0)
    return plus.astype(jnp.float32) - minus.astype(jnp.float32)


def _tri_matrix():
    r = lax.broadcasted_iota(jnp.int32, (SEQ, SEQ), 0)
    c = lax.broadcasted_iota(jnp.int32, (SEQ, SEQ), 1)
    return (c <= r).astype(jnp.float32)


def _work_items(starts):
    i32 = jnp.int32
    ends = jnp.concatenate([starts[1:], jnp.array([NSLOTS], i32)])
    nonempty = ends > starts
    first_t = starts // ROW_TILE
    last_t = jnp.where(nonempty, (ends - 1) // ROW_TILE, first_t)
    ntiles = jnp.where(nonempty, last_t - first_t + 1, 0)
    base = jnp.concatenate([jnp.zeros((1,), i32), jnp.cumsum(ntiles)])
    s = jnp.arange(MAX_ITEMS, dtype=i32)
    e = jnp.sum((s[:, None] >= base[None, 1:]).astype(i32), axis=1)
    e = jnp.clip(e, 0, NUM_EXPERTS - 1)
    valid = s < base[NUM_EXPERTS]
    tile = first_t[e] + (s - base[e])
    tile_id = jnp.where(valid, tile, N_TILES - 1).astype(i32)
    expert_id = jnp.where(valid, e, 0).astype(i32)
    row_start = jnp.where(valid, starts[e], 0).astype(i32)
    row_end = jnp.where(valid, ends[e], 0).astype(i32)
    first_flag = jnp.concatenate(
        [jnp.ones((1,), i32),
         (tile_id[1:] != tile_id[:-1]).astype(i32)])
    return tile_id, expert_id, row_start, row_end, first_flag


def kernel(x, Wqkv, Wo, Wgate, Weg, Weu, Wed, ln1, ln2):
    B, S, D = x.shape
    x2 = x.reshape(S, D)
    cos, sin = _rope_tables()
    rot = _rot_matrix()
    tri = _tri_matrix()

    qkv = _qkv_call(x2, Wqkv, ln1.reshape(1, EMB))
    q = qkv[:, :EMB].reshape(S, HEADS, HEAD_DIM).transpose(1, 0, 2)
    k = qkv[:, EMB:2 * EMB].reshape(S, HEADS, HEAD_DIM).transpose(1, 0, 2)
    v = qkv[:, 2 * EMB:].reshape(S, HEADS, HEAD_DIM).transpose(1, 0, 2)
    ao = _attn_call(q, k, v, cos, sin, rot)
    ao2 = ao.transpose(1, 0, 2).reshape(S, D)
    h1, m = _proj_call(x2, ao2, Wo, ln2.reshape(1, EMB))

    d0b, d1b, w0b, w1b, offb, auxb = _router_call(m, Wgate, tri)
    starts = offb[:, 0]
    tile_id, expert_id, row_start, row_end, first_flag = _work_items(starts)

    xs = _dispatch_sc(m, d0b, d1b)
    ffn = _ffn_call(tile_id, expert_id, row_start, row_end, first_flag,
                    xs, Weg, Weu, Wed)
    ga, gb = _gather_sc(ffn, d0b, d1b)
    out = _combine_call(h1, ga, gb, w0b, w1b)

    aux = auxb[0, 0]
    return out.reshape(B, S, D), aux
